# Initial kernel scaffold; baseline (speedup 1.0000x reference)
import jax
import jax.numpy as jnp
from jax.experimental import pallas as pl


def kernel(x, edge_index, edge_attr, batch, W_rel1, b_rel1, W_root1, W_rel2, b_rel2, W_root2, W_rel3, b_rel3, W_root3, W_rel4, b_rel4, W_root4, W_rel5, b_rel5, W_root5, W_lin, b_lin):
    raise NotImplementedError("write your pallas kernel here")


# speedup vs baseline: 7.7248x; 7.7248x over previous
import dataclasses
import functools

import jax
import jax.numpy as jnp
from jax import lax
from jax.experimental import pallas as pl
from jax.experimental.pallas import tpu as pltpu
from jax.experimental.pallas import tpu_sc as plsc

_N = 10000
_G = 64
_E = 320000
_C = 2
_NCORES = 2
_NSUB = 16
_NW = _NCORES * _NSUB
_BLK = 128
_NBLK = 79
_EW = _BLK * _NBLK
_EPAD = _EW * _NW
_NPAD = 10240
_ROWS_OUT = _NPAD // _NSUB


def _sc_compiler_params():
    cp = pltpu.CompilerParams()
    fields = pltpu.CompilerParams.__dataclass_fields__
    if "needs_layout_passes" in fields:
        cp = dataclasses.replace(cp, needs_layout_passes=False)
    if "use_tc_tiling_on_sc" in fields:
        cp = dataclasses.replace(cp, use_tc_tiling_on_sc=False)
    return cp


@functools.lru_cache(maxsize=None)
def _seg_scatter(d: int):
    ld = d.bit_length() - 1
    mesh = plsc.VectorSubcoreMesh(core_axis_name="c", subcore_axis_name="s")

    @functools.partial(
        pl.kernel,
        out_type=jax.ShapeDtypeStruct((_NCORES, _NPAD, d), jnp.float32),
        mesh=mesh,
        compiler_params=_sc_compiler_params(),
        scratch_types=[
            pltpu.VMEM((_NBLK, _BLK), jnp.int32),
            pltpu.VMEM((_NBLK, _BLK), jnp.int32),
            pltpu.VMEM((_NBLK, _BLK), jnp.float32),
            pltpu.VMEM((_BLK, d), jnp.float32),
            pltpu.VMEM_SHARED((_NPAD, d), jnp.float32),
            pltpu.SemaphoreType.DMA,
        ],
    )
    def k(h_hbm, src_hbm, dst_hbm, w_hbm, out_hbm,
          src_v, dst_v, w_v, rows_v, agg_sh, sem):
        c = lax.axis_index("c")
        s = lax.axis_index("s")
        wid = c * _NSUB + s
        iota = lax.iota(jnp.int32, 16)
        zeros = jnp.zeros((16,), jnp.float32)

        @pl.loop(0, _BLK * d, step=16)
        def _(p):
            v = p + iota
            plsc.store_scatter(rows_v, [v >> ld, v & (d - 1)], zeros)

        @pl.loop(0, _ROWS_OUT // _BLK)
        def _(i):
            pltpu.sync_copy(rows_v,
                            agg_sh.at[pl.ds(s * _ROWS_OUT + i * _BLK, _BLK)])

        plsc.subcore_barrier()

        pltpu.sync_copy(src_hbm.at[wid], src_v)
        pltpu.sync_copy(dst_hbm.at[wid], dst_v)
        pltpu.sync_copy(w_hbm.at[wid], w_v)

        @pl.loop(0, _NBLK)
        def _(j):
            pltpu.async_copy(h_hbm.at[src_v.at[j]], rows_v, sem).wait()
            j_splat = lax.full((16,), j, jnp.int32)
            if d >= 16:
                @pl.loop(0, _BLK)
                def _(e):
                    wv = plsc.load_gather(
                        w_v, [j_splat, lax.full((16,), e, jnp.int32)])
                    for kk in range(d // 16):
                        rv = rows_v[e, pl.ds(kk * 16, 16)]
                        rows_v[e, pl.ds(kk * 16, 16)] = rv * wv
            else:
                @pl.loop(0, _BLK * d, step=16)
                def _(p):
                    v = p + iota
                    ee = v >> ld
                    cc = v & (d - 1)
                    wv = plsc.load_gather(w_v, [j_splat, ee])
                    rv = plsc.load_gather(rows_v, [ee, cc])
                    plsc.store_scatter(rows_v, [ee, cc], rv * wv)
            pltpu.sync_copy(rows_v, agg_sh.at[dst_v.at[j]], add=True)

        plsc.subcore_barrier()
        pltpu.sync_copy(agg_sh.at[pl.ds(s * _ROWS_OUT, _ROWS_OUT)],
                        out_hbm.at[c, pl.ds(s * _ROWS_OUT, _ROWS_OUT)])

    return k


def _dot(a, b):
    return jnp.dot(a, b, preferred_element_type=jnp.float32,
                   precision=lax.Precision.HIGHEST)


def _mm_body(x_ref, w_ref, o_ref):
    o_ref[...] = _dot(x_ref[...], w_ref[...])


def _b1_body(a_ref, x_ref, br_ref, wt_ref, o_ref):
    agg = a_ref[0, :_N, :] + a_ref[1, :_N, :]
    o_ref[...] = agg + br_ref[...] + _dot(x_ref[...], wt_ref[...])


def _bi_body(a_ref, h_ref, wr_ref, br_ref, wt_ref, o_ref):
    agg = a_ref[0, :_N, :] + a_ref[1, :_N, :]
    o_ref[...] = (_dot(agg, wr_ref[...]) + br_ref[...]
                  + _dot(h_ref[...], wt_ref[...]))


def _fin_body(a_ref, h_ref, wr_ref, br_ref, wt_ref, batch_ref,
              wl_ref, bl_ref, o_ref):
    agg = a_ref[0, :_N, :] + a_ref[1, :_N, :]
    h5 = (_dot(agg, wr_ref[...]) + br_ref[...]
          + _dot(h_ref[...], wt_ref[...]))
    sel = (batch_ref[...] == lax.broadcasted_iota(jnp.int32, (_G, _N), 0))
    pooled = _dot(sel.astype(jnp.float32), h5)
    o_ref[...] = _dot(pooled, wl_ref[...]) + bl_ref[...]


def _tc(body, out_shape, *args):
    return pl.pallas_call(
        body, out_shape=jax.ShapeDtypeStruct(out_shape, jnp.float32))(*args)


def kernel(x, edge_index, edge_attr, batch,
           W_rel1, b_rel1, W_root1,
           W_rel2, b_rel2, W_root2,
           W_rel3, b_rel3, W_root3,
           W_rel4, b_rel4, W_root4,
           W_rel5, b_rel5, W_root5,
           W_lin, b_lin):
    f32 = jnp.float32
    src = edge_index[0]
    dst = edge_index[1]
    pad = _EPAD - _E
    srcp = jnp.concatenate([src, jnp.zeros((pad,), jnp.int32)]
                           ).reshape(_NW, _NBLK, _BLK)
    dstp = jnp.concatenate([dst, jnp.full((pad,), _N, jnp.int32)]
                           ).reshape(_NW, _NBLK, _BLK)
    wp = jnp.concatenate([edge_attr, jnp.zeros((pad,), f32)]
                         ).reshape(_NW, _NBLK, _BLK)
    batch2 = batch.reshape(1, _N)

    hr1 = _tc(_mm_body, (_N, 8), x, W_rel1)
    agg1 = _seg_scatter(8)(hr1, srcp, dstp, wp)
    h1 = _tc(_b1_body, (_N, 8), agg1, x, b_rel1.reshape(1, -1), W_root1)
    agg2 = _seg_scatter(8)(h1, srcp, dstp, wp)
    h2 = _tc(_bi_body, (_N, 16), agg2, h1, W_rel2, b_rel2.reshape(1, -1),
             W_root2)
    agg3 = _seg_scatter(16)(h2, srcp, dstp, wp)
    h3 = _tc(_bi_body, (_N, 32), agg3, h2, W_rel3, b_rel3.reshape(1, -1),
             W_root3)
    agg4 = _seg_scatter(32)(h3, srcp, dstp, wp)
    h4 = _tc(_bi_body, (_N, 64), agg4, h3, W_rel4, b_rel4.reshape(1, -1),
             W_root4)
    agg5 = _seg_scatter(64)(h4, srcp, dstp, wp)
    out = _tc(_fin_body, (_G, _C), agg5, h4, W_rel5, b_rel5.reshape(1, -1),
              W_root5, batch2, W_lin, b_lin.reshape(1, -1))
    return out
